# Initial kernel scaffold; baseline (speedup 1.0000x reference)
import functools

import jax
import jax.numpy as jnp
from jax import lax
from jax.experimental import pallas as pl
from jax.experimental.pallas import tpu as pltpu
from jax.experimental.pallas import tpu_sc as plsc

N_NODES = 10000
D = 128

NC = 2
NS = 16
NW = NC * NS

LANES = 128
G_PER_TILE = 79
E_PAD = NW * G_PER_TILE * LANES

N_PAD = 10240
ROWS_PER_TILE = N_PAD // NS
WB_CHUNKS = ROWS_PER_TILE // LANES


def _mm1_body(x_ref, w_ref, o_ref):
    o_ref[...] = jnp.dot(x_ref[...], w_ref[...],
                         preferred_element_type=jnp.float32)


def _mm1(x, W):
    BM = 500
    return pl.pallas_call(
        _mm1_body,
        grid=(N_NODES // BM,),
        in_specs=[
            pl.BlockSpec((BM, D), lambda i: (i, 0)),
            pl.BlockSpec((D, D), lambda i: (0, 0)),
        ],
        out_specs=pl.BlockSpec((BM, D), lambda i: (i, 0)),
        out_shape=jax.ShapeDtypeStruct((N_NODES, D), jnp.float32),
    )(x, W)


def _mm2_body(p0_ref, p1_ref, b_ref, w_ref, o_ref):
    h = jnp.maximum(p0_ref[...] + p1_ref[...] + b_ref[...], 0.0)
    o_ref[...] = jnp.dot(h, w_ref[...], preferred_element_type=jnp.float32)


def _mm2(partials, b, W):
    BM = 512
    nblk = N_PAD // BM
    return pl.pallas_call(
        _mm2_body,
        grid=(nblk,),
        in_specs=[
            pl.BlockSpec((BM, D), lambda i: (i, 0)),
            pl.BlockSpec((BM, D), lambda i: (i + nblk, 0)),
            pl.BlockSpec((1, D), lambda i: (0, 0)),
            pl.BlockSpec((D, D), lambda i: (0, 0)),
        ],
        out_specs=pl.BlockSpec((BM, D), lambda i: (i, 0)),
        out_shape=jax.ShapeDtypeStruct((N_PAD, D), jnp.float32),
    )(partials, partials, b.reshape(1, D), W)


def _final_body(q0_ref, q1_ref, b_ref, o_ref):
    o_ref[...] = q0_ref[...] + q1_ref[...] + b_ref[...]


def _final(partials, b):
    BM = 80
    return pl.pallas_call(
        _final_body,
        grid=(N_NODES // BM,),
        in_specs=[
            pl.BlockSpec((BM, D), lambda i: (i, 0)),
            pl.BlockSpec((BM, D), lambda i: (i + N_PAD // BM, 0)),
            pl.BlockSpec((1, D), lambda i: (0, 0)),
        ],
        out_specs=pl.BlockSpec((BM, D), lambda i: (i, 0)),
        out_shape=jax.ShapeDtypeStruct((N_NODES, D), jnp.float32),
    )(partials, partials, b.reshape(1, D))


def _sc_agg_body(sup_hbm, src_hbm, dst_hbm, out_hbm,
                 src_v, dst_v, rows_v, acc_sh, sem):
    cid = lax.axis_index("c")
    sid = lax.axis_index("s")
    wid = sid * NC + cid

    zero16 = jnp.zeros((16,), jnp.float32)

    def _zrow(r, carry):
        def _zcol(c, carry2):
            rows_v[r, pl.ds(c * 16, 16)] = zero16
            return carry2
        return lax.fori_loop(0, D // 16, _zcol, carry)

    lax.fori_loop(0, LANES, _zrow, 0)

    row0 = sid * ROWS_PER_TILE

    def _zcp(m, carry):
        pltpu.sync_copy(rows_v, acc_sh.at[pl.ds(row0 + m * LANES, LANES)])
        return carry

    lax.fori_loop(0, WB_CHUNKS, _zcp, 0)
    plsc.subcore_barrier()

    gbase = wid * G_PER_TILE
    pltpu.sync_copy(src_hbm.at[pl.ds(gbase, G_PER_TILE)], src_v)
    pltpu.sync_copy(dst_hbm.at[pl.ds(gbase, G_PER_TILE)], dst_v)

    def _step(g, carry):
        pltpu.async_copy(sup_hbm.at[src_v.at[g]], rows_v, sem).wait()
        pltpu.sync_copy(rows_v, acc_sh.at[dst_v.at[g]], add=True)
        return carry

    lax.fori_loop(0, G_PER_TILE, _step, 0)
    plsc.subcore_barrier()

    out_base = cid * N_PAD + row0

    def _wb(m, carry):
        pltpu.sync_copy(acc_sh.at[pl.ds(row0 + m * LANES, LANES)],
                        rows_v)
        pltpu.sync_copy(rows_v,
                        out_hbm.at[pl.ds(out_base + m * LANES, LANES)])
        return carry

    lax.fori_loop(0, WB_CHUNKS, _wb, 0)


def _sc_agg(support, src2d, dst2d):
    mesh = plsc.VectorSubcoreMesh(core_axis_name="c", subcore_axis_name="s",
                                  num_cores=NC, num_subcores=NS)
    kern = pl.kernel(
        _sc_agg_body,
        out_type=jax.ShapeDtypeStruct((NC * N_PAD, D), jnp.float32),
        mesh=mesh,
        scratch_types=[
            pltpu.VMEM((G_PER_TILE, LANES), jnp.int32),
            pltpu.VMEM((G_PER_TILE, LANES), jnp.int32),
            pltpu.VMEM((LANES, D), jnp.float32),
            pltpu.VMEM_SHARED((N_PAD, D), jnp.float32),
            pltpu.SemaphoreType.DMA,
        ],
    )
    return kern(support, src2d, dst2d)


def kernel(x, adj, W1, b1, W2, b2):
    src = adj[0].astype(jnp.int32)
    dst = adj[1].astype(jnp.int32)
    n_edges = src.shape[0]
    pad = E_PAD - n_edges
    src_p = jnp.concatenate(
        [src, jnp.zeros((pad,), jnp.int32)]).reshape(NW * G_PER_TILE, LANES)
    dst_p = jnp.concatenate(
        [dst, jnp.full((pad,), N_NODES, jnp.int32)]).reshape(
            NW * G_PER_TILE, LANES)

    support1 = _mm1(x, W1)
    p1 = _sc_agg(support1, src_p, dst_p)
    support2 = _mm2(p1, b1, W2)
    p2 = _sc_agg(support2, src_p, dst_p)
    return _final(p2, b2)


# speedup vs baseline: 3.1672x; 3.1672x over previous
import functools

import jax
import jax.numpy as jnp
from jax import lax
from jax.experimental import pallas as pl
from jax.experimental.pallas import tpu as pltpu
from jax.experimental.pallas import tpu_sc as plsc

N_NODES = 10000
D = 128

NC = 2
NS = 16
NW = NC * NS

LANES = 128
G_PER_TILE = 80
E_PAD = NW * G_PER_TILE * LANES

N_PAD = 10240
ROWS_PER_TILE = N_PAD // NS
WB_CHUNKS = ROWS_PER_TILE // LANES


def _mm1_body(x_ref, w_ref, o_ref):
    o_ref[...] = jnp.dot(x_ref[...], w_ref[...],
                         preferred_element_type=jnp.float32)


def _mm1(x, W):
    BM = 400
    return pl.pallas_call(
        _mm1_body,
        grid=(N_NODES // BM,),
        in_specs=[
            pl.BlockSpec((BM, D), lambda i: (i, 0)),
            pl.BlockSpec((D, D), lambda i: (0, 0)),
        ],
        out_specs=pl.BlockSpec((BM, D), lambda i: (i, 0)),
        out_shape=jax.ShapeDtypeStruct((N_NODES, D), jnp.float32),
    )(x, W)


def _mm2_body(p0_ref, p1_ref, b_ref, w_ref, o_ref):
    h = jnp.maximum(p0_ref[...] + p1_ref[...] + b_ref[...], 0.0)
    o_ref[...] = jnp.dot(h, w_ref[...], preferred_element_type=jnp.float32)


def _mm2(partials, b, W):
    BM = 512
    nblk = N_PAD // BM
    return pl.pallas_call(
        _mm2_body,
        grid=(nblk,),
        in_specs=[
            pl.BlockSpec((BM, D), lambda i: (i, 0)),
            pl.BlockSpec((BM, D), lambda i: (i + nblk, 0)),
            pl.BlockSpec((1, D), lambda i: (0, 0)),
            pl.BlockSpec((D, D), lambda i: (0, 0)),
        ],
        out_specs=pl.BlockSpec((BM, D), lambda i: (i, 0)),
        out_shape=jax.ShapeDtypeStruct((N_PAD, D), jnp.float32),
    )(partials, partials, b.reshape(1, D), W)


def _final_body(q0_ref, q1_ref, b_ref, o_ref):
    o_ref[...] = q0_ref[...] + q1_ref[...] + b_ref[...]


def _final(partials, b):
    BM = 80
    return pl.pallas_call(
        _final_body,
        grid=(N_NODES // BM,),
        in_specs=[
            pl.BlockSpec((BM, D), lambda i: (i, 0)),
            pl.BlockSpec((BM, D), lambda i: (i + N_PAD // BM, 0)),
            pl.BlockSpec((1, D), lambda i: (0, 0)),
        ],
        out_specs=pl.BlockSpec((BM, D), lambda i: (i, 0)),
        out_shape=jax.ShapeDtypeStruct((N_NODES, D), jnp.float32),
    )(partials, partials, b.reshape(1, D))


def _sc_agg_body(sup_hbm, src_hbm, dst_hbm, out_hbm,
                 src_v, dst_v, rows_v, acc_sh, sem):
    cid = lax.axis_index("c")
    sid = lax.axis_index("s")
    wid = sid * NC + cid

    zero16 = jnp.zeros((16,), jnp.float32)

    def _zrow(r, carry):
        def _zcol(c, carry2):
            rows_v[r, pl.ds(c * 16, 16)] = zero16
            return carry2
        return lax.fori_loop(0, D // 16, _zcol, carry)

    lax.fori_loop(0, LANES, _zrow, 0)

    row0 = sid * ROWS_PER_TILE

    def _zcp(m, carry):
        pltpu.sync_copy(rows_v, acc_sh.at[pl.ds(row0 + m * LANES, LANES)])
        return carry

    lax.fori_loop(0, WB_CHUNKS, _zcp, 0)
    plsc.subcore_barrier()

    gbase = wid * G_PER_TILE
    pltpu.sync_copy(src_hbm.at[pl.ds(gbase, G_PER_TILE)], src_v)
    pltpu.sync_copy(dst_hbm.at[pl.ds(gbase, G_PER_TILE)], dst_v)

    def _step(g, carry):
        pltpu.async_copy(sup_hbm.at[src_v.at[g]], rows_v, sem).wait()
        pltpu.sync_copy(rows_v, acc_sh.at[dst_v.at[g]], add=True)
        return carry

    lax.fori_loop(0, G_PER_TILE, _step, 0)
    plsc.subcore_barrier()

    out_base = cid * N_PAD + row0

    def _wb(m, carry):
        pltpu.sync_copy(acc_sh.at[pl.ds(row0 + m * LANES, LANES)],
                        rows_v)
        pltpu.sync_copy(rows_v,
                        out_hbm.at[pl.ds(out_base + m * LANES, LANES)])
        return carry

    lax.fori_loop(0, WB_CHUNKS, _wb, 0)


def _sc_agg(support, src2d, dst2d):
    mesh = plsc.VectorSubcoreMesh(core_axis_name="c", subcore_axis_name="s",
                                  num_cores=NC, num_subcores=NS)
    kern = pl.kernel(
        _sc_agg_body,
        out_type=jax.ShapeDtypeStruct((NC * N_PAD, D), jnp.float32),
        mesh=mesh,
        scratch_types=[
            pltpu.VMEM((G_PER_TILE, LANES), jnp.int32),
            pltpu.VMEM((G_PER_TILE, LANES), jnp.int32),
            pltpu.VMEM((LANES, D), jnp.float32),
            pltpu.VMEM_SHARED((N_PAD, D), jnp.float32),
            pltpu.SemaphoreType.DMA,
        ],
    )
    return kern(support, src2d, dst2d)


def kernel(x, adj, W1, b1, W2, b2):
    src = adj[0].astype(jnp.int32)
    dst = adj[1].astype(jnp.int32)
    n_edges = src.shape[0]
    pad = E_PAD - n_edges
    src_p = jnp.concatenate(
        [src, jnp.zeros((pad,), jnp.int32)]).reshape(NW * G_PER_TILE, LANES)
    dst_p = jnp.concatenate(
        [dst, jnp.full((pad,), N_NODES, jnp.int32)]).reshape(
            NW * G_PER_TILE, LANES)

    support1 = _mm1(x, W1)
    p1 = _sc_agg(support1, src_p, dst_p)
    support2 = _mm2(p1, b1, W2)
    p2 = _sc_agg(support2, src_p, dst_p)
    return _final(p2, b2)


# speedup vs baseline: 3.5369x; 1.1167x over previous
import functools

import jax
import jax.numpy as jnp
from jax import lax
from jax.experimental import pallas as pl
from jax.experimental.pallas import tpu as pltpu
from jax.experimental.pallas import tpu_sc as plsc

N_NODES = 10000
D = 128

NC = 2
NS = 16
NW = NC * NS

LANES = 128
G_PER_TILE = 80
E_PAD = NW * G_PER_TILE * LANES

N_PAD = 10240
ROWS_PER_TILE = N_PAD // NS
WB_CHUNKS = ROWS_PER_TILE // LANES
IDX_CHUNK = 16


def _mm1_body(x_ref, w_ref, o_ref):
    o_ref[...] = jnp.dot(x_ref[...], w_ref[...],
                         preferred_element_type=jnp.float32)


def _mm1(x, W):
    BM = 400
    return pl.pallas_call(
        _mm1_body,
        grid=(N_NODES // BM,),
        in_specs=[
            pl.BlockSpec((BM, D), lambda i: (i, 0)),
            pl.BlockSpec((D, D), lambda i: (0, 0)),
        ],
        out_specs=pl.BlockSpec((BM, D), lambda i: (i, 0)),
        out_shape=jax.ShapeDtypeStruct((N_NODES, D), jnp.float32),
    )(x, W)


def _mm2_body(p0_ref, p1_ref, b_ref, w_ref, o_ref):
    h = jnp.maximum(p0_ref[...] + p1_ref[...] + b_ref[...], 0.0)
    o_ref[...] = jnp.dot(h, w_ref[...], preferred_element_type=jnp.float32)


def _mm2(partials, b, W):
    BM = 512
    nblk = N_PAD // BM
    return pl.pallas_call(
        _mm2_body,
        grid=(nblk,),
        in_specs=[
            pl.BlockSpec((BM, D), lambda i: (i, 0)),
            pl.BlockSpec((BM, D), lambda i: (i + nblk, 0)),
            pl.BlockSpec((1, D), lambda i: (0, 0)),
            pl.BlockSpec((D, D), lambda i: (0, 0)),
        ],
        out_specs=pl.BlockSpec((BM, D), lambda i: (i, 0)),
        out_shape=jax.ShapeDtypeStruct((N_PAD, D), jnp.float32),
    )(partials, partials, b.reshape(1, D), W)


def _final_body(q0_ref, q1_ref, b_ref, o_ref):
    o_ref[...] = q0_ref[...] + q1_ref[...] + b_ref[...]


def _final(partials, b):
    BM = 80
    return pl.pallas_call(
        _final_body,
        grid=(N_NODES // BM,),
        in_specs=[
            pl.BlockSpec((BM, D), lambda i: (i, 0)),
            pl.BlockSpec((BM, D), lambda i: (i + N_PAD // BM, 0)),
            pl.BlockSpec((1, D), lambda i: (0, 0)),
        ],
        out_specs=pl.BlockSpec((BM, D), lambda i: (i, 0)),
        out_shape=jax.ShapeDtypeStruct((N_NODES, D), jnp.float32),
    )(partials, partials, b.reshape(1, D))


def _sc_agg_body(sup_hbm, src_hbm, dst_hbm, out_hbm,
                 src_v, dst_v, rows_v, rows_b, acc_sh, sem, sem_b):
    cid = lax.axis_index("c")
    sid = lax.axis_index("s")
    wid = sid * NC + cid

    zero16 = jnp.zeros((16,), jnp.float32)

    def _zrow(r, carry):
        def _zcol(c, carry2):
            rows_v[r, pl.ds(c * 16, 16)] = zero16
            return carry2
        return lax.fori_loop(0, D // 16, _zcol, carry)

    lax.fori_loop(0, LANES, _zrow, 0)

    row0 = sid * ROWS_PER_TILE

    def _zcp(m, carry):
        pltpu.sync_copy(rows_v, acc_sh.at[pl.ds(row0 + m * LANES, LANES)])
        return carry

    lax.fori_loop(0, WB_CHUNKS, _zcp, 0)
    plsc.subcore_barrier()

    gbase = wid * G_PER_TILE

    def _chunk(c, carry):
        base = gbase + c * IDX_CHUNK
        pltpu.sync_copy(src_hbm.at[pl.ds(base, IDX_CHUNK)], src_v)
        pltpu.sync_copy(dst_hbm.at[pl.ds(base, IDX_CHUNK)], dst_v)
        pltpu.async_copy(sup_hbm.at[src_v.at[0]], rows_v, sem)

        def _pair(t, carry2):
            g0 = 2 * t
            g1 = g0 + 1
            pltpu.async_copy(sup_hbm.at[src_v.at[g1]], rows_b, sem_b)
            pltpu.make_async_copy(sup_hbm.at[src_v.at[g0]], rows_v,
                                  sem).wait()
            pltpu.sync_copy(rows_v, acc_sh.at[dst_v.at[g0]], add=True)

            @pl.when(g1 + 1 < IDX_CHUNK)
            def _():
                pltpu.async_copy(sup_hbm.at[src_v.at[g1 + 1]], rows_v, sem)

            pltpu.make_async_copy(sup_hbm.at[src_v.at[g1]], rows_b,
                                  sem_b).wait()
            pltpu.sync_copy(rows_b, acc_sh.at[dst_v.at[g1]], add=True)
            return carry2

        lax.fori_loop(0, IDX_CHUNK // 2, _pair, 0)
        return carry

    lax.fori_loop(0, G_PER_TILE // IDX_CHUNK, _chunk, 0)
    plsc.subcore_barrier()

    out_base = cid * N_PAD + row0

    def _wb(m, carry):
        pltpu.sync_copy(acc_sh.at[pl.ds(row0 + m * LANES, LANES)],
                        rows_v)
        pltpu.sync_copy(rows_v,
                        out_hbm.at[pl.ds(out_base + m * LANES, LANES)])
        return carry

    lax.fori_loop(0, WB_CHUNKS, _wb, 0)


def _sc_agg(support, src2d, dst2d):
    mesh = plsc.VectorSubcoreMesh(core_axis_name="c", subcore_axis_name="s",
                                  num_cores=NC, num_subcores=NS)
    kern = pl.kernel(
        _sc_agg_body,
        out_type=jax.ShapeDtypeStruct((NC * N_PAD, D), jnp.float32),
        mesh=mesh,
        scratch_types=[
            pltpu.VMEM((IDX_CHUNK, LANES), jnp.int32),
            pltpu.VMEM((IDX_CHUNK, LANES), jnp.int32),
            pltpu.VMEM((LANES, D), jnp.float32),
            pltpu.VMEM((LANES, D), jnp.float32),
            pltpu.VMEM_SHARED((N_PAD, D), jnp.float32),
            pltpu.SemaphoreType.DMA,
            pltpu.SemaphoreType.DMA,
        ],
    )
    return kern(support, src2d, dst2d)


def kernel(x, adj, W1, b1, W2, b2):
    src = adj[0].astype(jnp.int32)
    dst = adj[1].astype(jnp.int32)
    n_edges = src.shape[0]
    pad = E_PAD - n_edges
    src_p = jnp.concatenate(
        [src, jnp.zeros((pad,), jnp.int32)]).reshape(NW * G_PER_TILE, LANES)
    dst_p = jnp.concatenate(
        [dst, jnp.full((pad,), N_NODES, jnp.int32)]).reshape(
            NW * G_PER_TILE, LANES)

    support1 = _mm1(x, W1)
    p1 = _sc_agg(support1, src_p, dst_p)
    support2 = _mm2(p1, b1, W2)
    p2 = _sc_agg(support2, src_p, dst_p)
    return _final(p2, b2)


# speedup vs baseline: 3.7303x; 1.0547x over previous
import functools

import jax
import jax.numpy as jnp
from jax import lax
from jax.experimental import pallas as pl
from jax.experimental.pallas import tpu as pltpu
from jax.experimental.pallas import tpu_sc as plsc

N_NODES = 10000
D = 128

NC = 2
NS = 16
NW = NC * NS

LANES = 128
G_TOTAL = 2560
E_PAD = G_TOTAL * LANES
G0_PER_TILE = 120
G1_PER_TILE = 40

N_PAD = 10240
ROWS_PER_TILE = N_PAD // NS
WB_CHUNKS = ROWS_PER_TILE // LANES
IDX_CHUNK = 8


def _mm1_body(x_ref, w_ref, o_ref):
    o_ref[...] = jnp.dot(x_ref[...], w_ref[...],
                         preferred_element_type=jnp.float32)


def _mm1(x, W):
    BM = 400
    return pl.pallas_call(
        _mm1_body,
        grid=(N_NODES // BM,),
        in_specs=[
            pl.BlockSpec((BM, D), lambda i: (i, 0)),
            pl.BlockSpec((D, D), lambda i: (0, 0)),
        ],
        out_specs=pl.BlockSpec((BM, D), lambda i: (i, 0)),
        out_shape=jax.ShapeDtypeStruct((N_NODES, D), jnp.float32),
    )(x, W)


def _mm2_body(p0_ref, p1_ref, b_ref, w_ref, o_ref):
    h = jnp.maximum(p0_ref[...] + p1_ref[...] + b_ref[...], 0.0)
    o_ref[...] = jnp.dot(h, w_ref[...], preferred_element_type=jnp.float32)


def _mm2(partials, b, W):
    BM = 512
    nblk = N_PAD // BM
    return pl.pallas_call(
        _mm2_body,
        grid=(nblk,),
        in_specs=[
            pl.BlockSpec((BM, D), lambda i: (i, 0)),
            pl.BlockSpec((BM, D), lambda i: (i + nblk, 0)),
            pl.BlockSpec((1, D), lambda i: (0, 0)),
            pl.BlockSpec((D, D), lambda i: (0, 0)),
        ],
        out_specs=pl.BlockSpec((BM, D), lambda i: (i, 0)),
        out_shape=jax.ShapeDtypeStruct((N_PAD, D), jnp.float32),
    )(partials, partials, b.reshape(1, D), W)


def _final_body(q0_ref, q1_ref, b_ref, o_ref):
    o_ref[...] = q0_ref[...] + q1_ref[...] + b_ref[...]


def _final(partials, b):
    BM = 80
    return pl.pallas_call(
        _final_body,
        grid=(N_NODES // BM,),
        in_specs=[
            pl.BlockSpec((BM, D), lambda i: (i, 0)),
            pl.BlockSpec((BM, D), lambda i: (i + N_PAD // BM, 0)),
            pl.BlockSpec((1, D), lambda i: (0, 0)),
        ],
        out_specs=pl.BlockSpec((BM, D), lambda i: (i, 0)),
        out_shape=jax.ShapeDtypeStruct((N_NODES, D), jnp.float32),
    )(partials, partials, b.reshape(1, D))


def _sc_agg_body(sup_hbm, src_hbm, dst_hbm, out_hbm,
                 src_v, dst_v, rows_v, rows_b, acc_sh, sem, sem_b):
    cid = lax.axis_index("c")
    sid = lax.axis_index("s")

    zero16 = jnp.zeros((16,), jnp.float32)

    def _zrow(r, carry):
        def _zcol(c, carry2):
            rows_v[r, pl.ds(c * 16, 16)] = zero16
            return carry2
        return lax.fori_loop(0, D // 16, _zcol, carry)

    lax.fori_loop(0, LANES, _zrow, 0)

    row0 = sid * ROWS_PER_TILE

    def _zcp(m, carry):
        pltpu.sync_copy(rows_v, acc_sh.at[pl.ds(row0 + m * LANES, LANES)])
        return carry

    lax.fori_loop(0, WB_CHUNKS, _zcp, 0)
    plsc.subcore_barrier()

    gbase = jnp.where(cid == 0, sid * G0_PER_TILE,
                      NS * G0_PER_TILE + sid * G1_PER_TILE)
    n_chunks = jnp.where(cid == 0, G0_PER_TILE // IDX_CHUNK,
                         G1_PER_TILE // IDX_CHUNK)

    def _chunk(c, carry):
        base = gbase + c * IDX_CHUNK
        pltpu.sync_copy(src_hbm.at[pl.ds(base, IDX_CHUNK)], src_v)
        pltpu.sync_copy(dst_hbm.at[pl.ds(base, IDX_CHUNK)], dst_v)
        pltpu.async_copy(sup_hbm.at[src_v.at[0]], rows_v, sem)

        def _pair(t, carry2):
            g0 = 2 * t
            g1 = g0 + 1
            pltpu.async_copy(sup_hbm.at[src_v.at[g1]], rows_b, sem_b)
            pltpu.make_async_copy(sup_hbm.at[src_v.at[g0]], rows_v,
                                  sem).wait()
            pltpu.sync_copy(rows_v, acc_sh.at[dst_v.at[g0]], add=True)

            @pl.when(g1 + 1 < IDX_CHUNK)
            def _():
                pltpu.async_copy(sup_hbm.at[src_v.at[g1 + 1]], rows_v, sem)

            pltpu.make_async_copy(sup_hbm.at[src_v.at[g1]], rows_b,
                                  sem_b).wait()
            pltpu.sync_copy(rows_b, acc_sh.at[dst_v.at[g1]], add=True)
            return carry2

        lax.fori_loop(0, IDX_CHUNK // 2, _pair, 0)
        return carry

    lax.fori_loop(0, n_chunks, _chunk, 0)
    plsc.subcore_barrier()

    out_base = cid * N_PAD + row0

    def _wb(m, carry):
        pltpu.sync_copy(acc_sh.at[pl.ds(row0 + m * LANES, LANES)],
                        rows_v)
        pltpu.sync_copy(rows_v,
                        out_hbm.at[pl.ds(out_base + m * LANES, LANES)])
        return carry

    lax.fori_loop(0, WB_CHUNKS, _wb, 0)


def _sc_agg(support, src2d, dst2d):
    mesh = plsc.VectorSubcoreMesh(core_axis_name="c", subcore_axis_name="s",
                                  num_cores=NC, num_subcores=NS)
    kern = pl.kernel(
        _sc_agg_body,
        out_type=jax.ShapeDtypeStruct((NC * N_PAD, D), jnp.float32),
        mesh=mesh,
        scratch_types=[
            pltpu.VMEM((IDX_CHUNK, LANES), jnp.int32),
            pltpu.VMEM((IDX_CHUNK, LANES), jnp.int32),
            pltpu.VMEM((LANES, D), jnp.float32),
            pltpu.VMEM((LANES, D), jnp.float32),
            pltpu.VMEM_SHARED((N_PAD, D), jnp.float32),
            pltpu.SemaphoreType.DMA,
            pltpu.SemaphoreType.DMA,
        ],
    )
    return kern(support, src2d, dst2d)


def kernel(x, adj, W1, b1, W2, b2):
    src = adj[0].astype(jnp.int32)
    dst = adj[1].astype(jnp.int32)
    n_edges = src.shape[0]
    pad = E_PAD - n_edges
    src_p = jnp.concatenate(
        [src, jnp.zeros((pad,), jnp.int32)]).reshape(G_TOTAL, LANES)
    dst_p = jnp.concatenate(
        [dst, jnp.full((pad,), N_NODES, jnp.int32)]).reshape(
            G_TOTAL, LANES)

    support1 = _mm1(x, W1)
    p1 = _sc_agg(support1, src_p, dst_p)
    support2 = _mm2(p1, b1, W2)
    p2 = _sc_agg(support2, src_p, dst_p)
    return _final(p2, b2)
